# speedup vs baseline: 4.8606x; 4.8606x over previous
import functools

import jax
import jax.numpy as jnp
from jax import lax
from jax.experimental import pallas as pl
from jax.experimental.pallas import tpu as pltpu
from jax.experimental.pallas import tpu_sc as plsc

VOCAB = 100000
EMBED = 128
BATCH = 1024
HIST = 200
TOTAL = BATCH * HIST

NC = 2
NS = 16
NW = NC * NS
B_PER_W = TOTAL // NW
CHUNK = 128
N_CHUNKS = B_PER_W // CHUNK

_mesh = plsc.VectorSubcoreMesh(core_axis_name="c", subcore_axis_name="s")


@functools.partial(
    pl.kernel,
    mesh=_mesh,
    out_type=jax.ShapeDtypeStruct((TOTAL, EMBED), jnp.float32),
    scratch_types=[
        pltpu.VMEM((CHUNK,), jnp.int32),
        pltpu.VMEM((CHUNK, EMBED), jnp.float32),
        pltpu.SemaphoreType.DMA,
    ],
)
def _gather_kernel(idx_hbm, table_hbm, out_hbm, idx_v, rows_v, sem):
    wid = lax.axis_index("s") * NC + lax.axis_index("c")

    def body(i, carry):
        base = wid * B_PER_W + i * CHUNK
        pltpu.sync_copy(idx_hbm.at[pl.ds(base, CHUNK)], idx_v)
        pltpu.async_copy(table_hbm.at[idx_v], rows_v, sem).wait()
        pltpu.sync_copy(rows_v, out_hbm.at[pl.ds(base, CHUNK)])
        return carry

    lax.fori_loop(0, N_CHUNKS, body, 0)


def kernel(input, table):
    idx = input.reshape(TOTAL).astype(jnp.int32)
    out = _gather_kernel(idx, table)
    return out.reshape(BATCH, HIST, EMBED)


# speedup vs baseline: 6.5887x; 1.3555x over previous
import functools

import jax
import jax.numpy as jnp
from jax import lax
from jax.experimental import pallas as pl
from jax.experimental.pallas import tpu as pltpu
from jax.experimental.pallas import tpu_sc as plsc

VOCAB = 100000
EMBED = 128
BATCH = 1024
HIST = 200
TOTAL = BATCH * HIST

NC = 2
NS = 16
NW = NC * NS
B_PER_W = TOTAL // NW
CHUNK = 128
N_CHUNKS = B_PER_W // CHUNK
N_PAIRS = N_CHUNKS // 2

_mesh = plsc.VectorSubcoreMesh(core_axis_name="c", subcore_axis_name="s")


@functools.partial(
    pl.kernel,
    mesh=_mesh,
    out_type=jax.ShapeDtypeStruct((TOTAL, EMBED), jnp.float32),
    scratch_types=[
        pltpu.VMEM((N_CHUNKS, CHUNK), jnp.int32),
        pltpu.VMEM((CHUNK, EMBED), jnp.float32),
        pltpu.VMEM((CHUNK, EMBED), jnp.float32),
        pltpu.SemaphoreType.DMA,
        pltpu.SemaphoreType.DMA,
        pltpu.SemaphoreType.DMA,
        pltpu.SemaphoreType.DMA,
    ],
)
def _gather_kernel(idx_hbm, table_hbm, out_hbm, idx_v, rows0, rows1,
                   sg0, sg1, so0, so1):
    wid = lax.axis_index("s") * NC + lax.axis_index("c")
    base = wid * B_PER_W

    def out_slc(c):
        return out_hbm.at[pl.ds(base + c * CHUNK, CHUNK)]

    pltpu.sync_copy(idx_hbm.at[wid], idx_v)

    pltpu.async_copy(table_hbm.at[idx_v.at[0]], rows0, sg0)

    def body(p, carry):
        c0 = 2 * p
        pltpu.make_async_copy(table_hbm.at[idx_v.at[0]], rows0, sg0).wait()

        @pl.when(p > 0)
        def _():
            pltpu.make_async_copy(rows1, out_slc(0), so1).wait()

        pltpu.async_copy(table_hbm.at[idx_v.at[c0 + 1]], rows1, sg1)
        pltpu.async_copy(rows0, out_slc(c0), so0)

        pltpu.make_async_copy(table_hbm.at[idx_v.at[0]], rows1, sg1).wait()
        pltpu.make_async_copy(rows0, out_slc(0), so0).wait()

        @pl.when(p < N_PAIRS - 1)
        def _():
            pltpu.async_copy(table_hbm.at[idx_v.at[c0 + 2]], rows0, sg0)

        pltpu.async_copy(rows1, out_slc(c0 + 1), so1)
        return carry

    lax.fori_loop(0, N_PAIRS, body, 0)

    pltpu.make_async_copy(rows1, out_slc(0), so1).wait()


def kernel(input, table):
    idx = input.reshape(TOTAL).astype(jnp.int32)
    out = _gather_kernel(idx.reshape(NW, N_CHUNKS, CHUNK), table)
    return out.reshape(BATCH, HIST, EMBED)


# speedup vs baseline: 7.6965x; 1.1681x over previous
import functools

import jax
import jax.numpy as jnp
from jax import lax
from jax.experimental import pallas as pl
from jax.experimental.pallas import tpu as pltpu
from jax.experimental.pallas import tpu_sc as plsc

VOCAB = 100000
EMBED = 128
BATCH = 1024
HIST = 200
TOTAL = BATCH * HIST

NC = 2
NS = 16
NW = NC * NS
B_PER_W = TOTAL // NW
CHUNK = 128
N_CHUNKS = B_PER_W // CHUNK
G = 2
ROWS_G = G * CHUNK
N_GROUPS = N_CHUNKS // G

_mesh = plsc.VectorSubcoreMesh(core_axis_name="c", subcore_axis_name="s")


@functools.partial(
    pl.kernel,
    mesh=_mesh,
    out_type=jax.ShapeDtypeStruct((TOTAL, EMBED), jnp.float32),
    scratch_types=[
        pltpu.VMEM((N_CHUNKS, CHUNK), jnp.int32),
        pltpu.VMEM((ROWS_G, EMBED), jnp.float32),
        pltpu.VMEM((ROWS_G, EMBED), jnp.float32),
        pltpu.SemaphoreType.DMA,
        pltpu.SemaphoreType.DMA,
        pltpu.SemaphoreType.DMA,
        pltpu.SemaphoreType.DMA,
    ],
)
def _gather_kernel(idx_hbm, table_hbm, out_hbm, idx_v, rows_a, rows_b,
                   sg_a, sg_b, so_a, so_b):
    wid = lax.axis_index("s") * NC + lax.axis_index("c")
    base = wid * B_PER_W

    pltpu.sync_copy(idx_hbm.at[wid], idx_v)

    def fire_gathers(g, rows, sg):
        for j in range(G):
            pltpu.async_copy(table_hbm.at[idx_v.at[g * G + j]],
                             rows.at[pl.ds(j * CHUNK, CHUNK)], sg)

    def wait_gathers(rows, sg):
        for j in range(G):
            pltpu.make_async_copy(table_hbm.at[idx_v.at[0]],
                                  rows.at[pl.ds(j * CHUNK, CHUNK)], sg).wait()

    def out_slc(g):
        return out_hbm.at[pl.ds(base + g * ROWS_G, ROWS_G)]

    def process(g, cur, cur_sg, cur_so, oth, oth_so):
        wait_gathers(cur, cur_sg)

        @pl.when(g > 0)
        def _():
            pltpu.make_async_copy(oth, out_slc(0), oth_so).wait()

        pltpu.async_copy(cur, out_slc(g), cur_so)

    fire_gathers(0, rows_a, sg_a)

    def body(g, carry):
        even = (g % 2) == 0

        @pl.when(even)
        def _():
            process(g, rows_a, sg_a, so_a, rows_b, so_b)

            @pl.when(g < N_GROUPS - 1)
            def _():
                fire_gathers(g + 1, rows_b, sg_b)

        @pl.when(jnp.logical_not(even))
        def _():
            process(g, rows_b, sg_b, so_b, rows_a, so_a)

            @pl.when(g < N_GROUPS - 1)
            def _():
                fire_gathers(g + 1, rows_a, sg_a)

        return carry

    lax.fori_loop(0, N_GROUPS, body, 0)

    pltpu.make_async_copy(rows_a, out_slc(0), so_a).wait()


def kernel(input, table):
    idx = input.reshape(TOTAL).astype(jnp.int32)
    out = _gather_kernel(idx.reshape(NW, N_CHUNKS, CHUNK), table)
    return out.reshape(BATCH, HIST, EMBED)


# speedup vs baseline: 8.0780x; 1.0496x over previous
import functools

import jax
import jax.numpy as jnp
from jax import lax
from jax.experimental import pallas as pl
from jax.experimental.pallas import tpu as pltpu
from jax.experimental.pallas import tpu_sc as plsc

VOCAB = 100000
EMBED = 128
BATCH = 1024
HIST = 200
TOTAL = BATCH * HIST

NC = 2
NS = 16
NW = NC * NS
B_PER_W = TOTAL // NW
CHUNK = 128
N_CHUNKS = B_PER_W // CHUNK
G = 2
ROWS_G = G * CHUNK
N_GROUPS = N_CHUNKS // G

_mesh = plsc.VectorSubcoreMesh(core_axis_name="c", subcore_axis_name="s")


@functools.partial(
    pl.kernel,
    mesh=_mesh,
    out_type=jax.ShapeDtypeStruct((TOTAL, EMBED), jnp.float32),
    scratch_types=[
        pltpu.VMEM((N_CHUNKS, CHUNK), jnp.int32),
        pltpu.VMEM((ROWS_G, EMBED), jnp.float32),
        pltpu.VMEM((ROWS_G, EMBED), jnp.float32),
        pltpu.VMEM((ROWS_G, EMBED), jnp.float32),
        pltpu.SemaphoreType.DMA,
        pltpu.SemaphoreType.DMA,
        pltpu.SemaphoreType.DMA,
        pltpu.SemaphoreType.DMA,
        pltpu.SemaphoreType.DMA,
        pltpu.SemaphoreType.DMA,
    ],
)
def _gather_kernel(idx_hbm, table_hbm, out_hbm, idx_v, rows_a, rows_b, rows_c,
                   sg_a, sg_b, sg_c, so_a, so_b, so_c):
    wid = lax.axis_index("s") * NC + lax.axis_index("c")
    base = wid * B_PER_W

    pltpu.sync_copy(idx_hbm.at[wid], idx_v)

    def fire_gathers(g, rows, sg):
        for j in range(G):
            pltpu.async_copy(table_hbm.at[idx_v.at[g * G + j]],
                             rows.at[pl.ds(j * CHUNK, CHUNK)], sg)

    def wait_gathers(rows, sg):
        for j in range(G):
            pltpu.make_async_copy(table_hbm.at[idx_v.at[0]],
                                  rows.at[pl.ds(j * CHUNK, CHUNK)], sg).wait()

    def out_slc(g):
        return out_hbm.at[pl.ds(base + g * ROWS_G, ROWS_G)]

    def step(g, cur, cur_sg, cur_so, oth, oth_sg, oth_so):
        wait_gathers(cur, cur_sg)

        @pl.when(g > 0)
        def _():
            pltpu.make_async_copy(oth, out_slc(0), oth_so).wait()

        pltpu.async_copy(cur, out_slc(g), cur_so)

        @pl.when(g < N_GROUPS - 2)
        def _():
            fire_gathers(g + 2, oth, oth_sg)

    fire_gathers(0, rows_a, sg_a)
    fire_gathers(1, rows_b, sg_b)

    def body(g, carry):
        r = g % 3

        @pl.when(r == 0)
        def _():
            step(g, rows_a, sg_a, so_a, rows_c, sg_c, so_c)

        @pl.when(r == 1)
        def _():
            step(g, rows_b, sg_b, so_b, rows_a, sg_a, so_a)

        @pl.when(r == 2)
        def _():
            step(g, rows_c, sg_c, so_c, rows_b, sg_b, so_b)

        return carry

    lax.fori_loop(0, N_GROUPS, body, 0)

    pltpu.make_async_copy(rows_a, out_slc(0), so_a).wait()


def kernel(input, table):
    idx = input.reshape(TOTAL).astype(jnp.int32)
    out = _gather_kernel(idx.reshape(NW, N_CHUNKS, CHUNK), table)
    return out.reshape(BATCH, HIST, EMBED)
